# Initial kernel scaffold; baseline (speedup 1.0000x reference)
import jax
import jax.numpy as jnp
from jax.experimental import pallas as pl


def kernel(x, edge_index, edge_weight, emb, W1, b1, W2, b2, ln_g, ln_b, mlp_W, mlp_b, out_W, out_b):
    raise NotImplementedError("write your pallas kernel here")


# speedup vs baseline: 2.0324x; 2.0324x over previous
import functools
import jax
import jax.numpy as jnp
from jax import lax
from jax.experimental import pallas as pl
from jax.experimental.pallas import tpu as pltpu

N = 10000
E = 320000
D = 128
V = 25
L = 6
EPS = 1e-7
BLK = 1000


def _encode_body(x_ref, emb_ref, maxw_ref, h_ref, c_ref):
    xb = x_ref[:]
    a = jnp.zeros((BLK, V), jnp.float32)
    for j in range(3):
        col = xb[:, j:j + 1]
        ids = lax.broadcasted_iota(jnp.int32, (BLK, V), 1)
        a = a + (ids == col).astype(jnp.float32)
    hb = jnp.dot(a, emb_ref[:], preferred_element_type=jnp.float32, precision=lax.Precision.HIGHEST)
    h_ref[:] = hb
    bm = jnp.max(hb, axis=0, keepdims=True)
    step = pl.program_id(0)

    @pl.when(step == 0)
    def _():
        c_ref[:] = bm

    @pl.when(step > 0)
    def _():
        c_ref[:] = jnp.maximum(c_ref[:], bm)

    @pl.when(step == pl.num_programs(0) - 1)
    def _():
        c_ref[:] = jnp.maximum(c_ref[:] + maxw_ref[:], 0.0) + EPS


def _encode(x, emb, maxw):
    grid = N // BLK
    return pl.pallas_call(
        _encode_body,
        grid=(grid,),
        in_specs=[
            pl.BlockSpec((BLK, 3), lambda i: (i, 0)),
            pl.BlockSpec((V, D), lambda i: (0, 0)),
            pl.BlockSpec((1, 1), lambda i: (0, 0)),
        ],
        out_specs=[
            pl.BlockSpec((BLK, D), lambda i: (i, 0)),
            pl.BlockSpec((1, D), lambda i: (0, 0)),
        ],
        out_shape=[
            jax.ShapeDtypeStruct((N, D), jnp.float32),
            jax.ShapeDtypeStruct((1, D), jnp.float32),
        ],
    )(x, emb, maxw)


def _maxw_body(w_ref, o_ref):
    o_ref[:] = jnp.max(w_ref[:]).reshape(1, 1)


def _maxw(ew):
    w2 = ew.reshape(2500, 128)
    return pl.pallas_call(
        _maxw_body,
        out_shape=jax.ShapeDtypeStruct((1, 1), jnp.float32),
    )(w2)


def _layer_body(h_ref, d0_ref, d1_ref, n0_ref, n1_ref, w1a_ref, w1b_ref,
                b1_ref, w2_ref, b2_ref, maxw_ref, h_out, c_ref):
    hb = h_ref[:]
    agg0 = n0_ref[:] / (d0_ref[:] + 1e-16)
    agg1 = n1_ref[:] / (d1_ref[:] + 1e-16)
    o0 = agg0 + hb[:, :64]
    o1 = agg1 + hb[:, 64:]
    z = (jnp.dot(o0, w1a_ref[:], preferred_element_type=jnp.float32, precision=lax.Precision.HIGHEST)
         + jnp.dot(o1, w1b_ref[:], preferred_element_type=jnp.float32, precision=lax.Precision.HIGHEST)
         + b1_ref[:])
    z = jnp.maximum(z, 0.0)
    hn = jnp.dot(z, w2_ref[:], preferred_element_type=jnp.float32, precision=lax.Precision.HIGHEST) + b2_ref[:]
    hn = jnp.maximum(hn, 0.0)
    h_out[:] = hn
    bm = jnp.max(hn, axis=0, keepdims=True)
    step = pl.program_id(0)

    @pl.when(step == 0)
    def _():
        c_ref[:] = bm

    @pl.when(step > 0)
    def _():
        c_ref[:] = jnp.maximum(c_ref[:], bm)

    @pl.when(step == pl.num_programs(0) - 1)
    def _():
        c_ref[:] = jnp.maximum(c_ref[:] + maxw_ref[:], 0.0) + EPS


def _layer_mlp(h, den0, den1, num0, num1, W1, b1, W2, b2, maxw):
    grid = N // BLK
    w1a, w1b = W1[:64], W1[64:]
    return pl.pallas_call(
        _layer_body,
        grid=(grid,),
        in_specs=[
            pl.BlockSpec((BLK, D), lambda i: (i, 0)),
            pl.BlockSpec((BLK, 64), lambda i: (i, 0)),
            pl.BlockSpec((BLK, 64), lambda i: (i, 0)),
            pl.BlockSpec((BLK, 64), lambda i: (i, 0)),
            pl.BlockSpec((BLK, 64), lambda i: (i, 0)),
            pl.BlockSpec((64, 2 * D), lambda i: (0, 0)),
            pl.BlockSpec((64, 2 * D), lambda i: (0, 0)),
            pl.BlockSpec((1, 2 * D), lambda i: (0, 0)),
            pl.BlockSpec((2 * D, D), lambda i: (0, 0)),
            pl.BlockSpec((1, D), lambda i: (0, 0)),
            pl.BlockSpec((1, 1), lambda i: (0, 0)),
        ],
        out_specs=[
            pl.BlockSpec((BLK, D), lambda i: (i, 0)),
            pl.BlockSpec((1, D), lambda i: (0, 0)),
        ],
        out_shape=[
            jax.ShapeDtypeStruct((N, D), jnp.float32),
            jax.ShapeDtypeStruct((1, D), jnp.float32),
        ],
    )(h, den0, den1, num0, num1, w1a, w1b, b1.reshape(1, -1), W2,
      b2.reshape(1, -1), maxw)


def _head_body(h_ref, g_ref, b_ref, mw_ref, mb_ref, ow_ref, ob_ref, o_ref):
    hb = h_ref[:]
    mu = jnp.mean(hb, axis=-1, keepdims=True)
    var = jnp.mean((hb - mu) ** 2, axis=-1, keepdims=True)
    hb = (hb - mu) * lax.rsqrt(var + 1e-5) * g_ref[:] + b_ref[:]
    for j in range(3):
        z = jnp.dot(hb, mw_ref[j], preferred_element_type=jnp.float32, precision=lax.Precision.HIGHEST) + mb_ref[j]
        hb = 0.5 * z * (1.0 + lax.erf(z * 0.7071067811865476))
    o = jnp.dot(hb, ow_ref[:], preferred_element_type=jnp.float32, precision=lax.Precision.HIGHEST) + ob_ref[:]
    o_ref[:] = o


def _head(h, ln_g, ln_b, mlp_W, mlp_b, out_W, out_b):
    grid = N // BLK
    return pl.pallas_call(
        _head_body,
        grid=(grid,),
        in_specs=[
            pl.BlockSpec((BLK, D), lambda i: (i, 0)),
            pl.BlockSpec((1, D), lambda i: (0, 0)),
            pl.BlockSpec((1, D), lambda i: (0, 0)),
            pl.BlockSpec((3, D, D), lambda i: (0, 0, 0)),
            pl.BlockSpec((3, 1, D), lambda i: (0, 0, 0)),
            pl.BlockSpec((D, 1), lambda i: (0, 0)),
            pl.BlockSpec((1, 1), lambda i: (0, 0)),
        ],
        out_specs=pl.BlockSpec((BLK, 1), lambda i: (i, 0)),
        out_shape=jax.ShapeDtypeStruct((N, 1), jnp.float32),
    )(h, ln_g.reshape(1, D), ln_b.reshape(1, D), mlp_W,
      mlp_b.reshape(3, 1, D), out_W, out_b.reshape(1, 1))


def _agg_sums_xla(h, src, dst, w, C):
    msg = jnp.maximum(h[src] + w, 0.0) + EPS
    p = jnp.exp(msg - C)
    den = jax.ops.segment_sum(p, dst, num_segments=N)
    num = jax.ops.segment_sum(msg * p, dst, num_segments=N)
    return (den[:, :64], den[:, 64:], num[:, :64], num[:, 64:])


def kernel(x, edge_index, edge_weight, emb, W1, b1, W2, b2, ln_g, ln_b,
           mlp_W, mlp_b, out_W, out_b):
    src = edge_index[0]
    dst = edge_index[1]
    w = edge_weight
    maxw = _maxw(edge_weight)
    h, C = _encode(x, emb, maxw)
    for i in range(L):
        den0, den1, num0, num1 = _agg_sums_xla(h, src, dst, w, C)
        h, C = _layer_mlp(h, den0, den1, num0, num1, W1[i], b1[i], W2[i],
                          b2[i], maxw)
    o = _head(h, ln_g, ln_b, mlp_W, mlp_b, out_W, out_b)
    return o[:, 0]


# speedup vs baseline: 3.3571x; 1.6518x over previous
import functools
import jax
import jax.numpy as jnp
from jax import lax
from jax.experimental import pallas as pl
from jax.experimental.pallas import tpu as pltpu

N = 10000
E = 320000
D = 128
V = 25
L = 6
EPS = 1e-7
BLK = 1000


def _encode_body(x_ref, emb_ref, maxw_ref, h_ref, c_ref):
    xb = x_ref[:]
    a = jnp.zeros((BLK, V), jnp.float32)
    for j in range(3):
        col = xb[:, j:j + 1]
        ids = lax.broadcasted_iota(jnp.int32, (BLK, V), 1)
        a = a + (ids == col).astype(jnp.float32)
    hb = jnp.dot(a, emb_ref[:], preferred_element_type=jnp.float32, precision=lax.Precision.HIGHEST)
    h_ref[:] = hb
    bm = jnp.max(hb, axis=0, keepdims=True)
    step = pl.program_id(0)

    @pl.when(step == 0)
    def _():
        c_ref[:] = bm

    @pl.when(step > 0)
    def _():
        c_ref[:] = jnp.maximum(c_ref[:], bm)

    @pl.when(step == pl.num_programs(0) - 1)
    def _():
        c_ref[:] = jnp.maximum(c_ref[:] + maxw_ref[:], 0.0) + EPS


def _encode(x, emb, maxw):
    grid = N // BLK
    return pl.pallas_call(
        _encode_body,
        grid=(grid,),
        in_specs=[
            pl.BlockSpec((BLK, 3), lambda i: (i, 0)),
            pl.BlockSpec((V, D), lambda i: (0, 0)),
            pl.BlockSpec((1, 1), lambda i: (0, 0)),
        ],
        out_specs=[
            pl.BlockSpec((BLK, D), lambda i: (i, 0)),
            pl.BlockSpec((1, D), lambda i: (0, 0)),
        ],
        out_shape=[
            jax.ShapeDtypeStruct((N, D), jnp.float32),
            jax.ShapeDtypeStruct((1, D), jnp.float32),
        ],
    )(x, emb, maxw)


def _maxw_body(w_ref, o_ref):
    o_ref[:] = jnp.max(w_ref[:]).reshape(1, 1)


def _maxw(ew):
    w2 = ew.reshape(2500, 128)
    return pl.pallas_call(
        _maxw_body,
        out_shape=jax.ShapeDtypeStruct((1, 1), jnp.float32),
    )(w2)


def _layer_body(h_ref, dn0_ref, dn1_ref, w1a_ref, w1b_ref,
                b1_ref, w2_ref, b2_ref, maxw_ref, h_out, c_ref):
    hb = h_ref[:]
    dn0 = dn0_ref[:]
    dn1 = dn1_ref[:]
    agg0 = dn0[:, 64:] / (dn0[:, :64] + 1e-16)
    agg1 = dn1[:, 64:] / (dn1[:, :64] + 1e-16)
    o0 = agg0 + hb[:, :64]
    o1 = agg1 + hb[:, 64:]
    z = (jnp.dot(o0, w1a_ref[:], preferred_element_type=jnp.float32, precision=lax.Precision.HIGHEST)
         + jnp.dot(o1, w1b_ref[:], preferred_element_type=jnp.float32, precision=lax.Precision.HIGHEST)
         + b1_ref[:])
    z = jnp.maximum(z, 0.0)
    hn = jnp.dot(z, w2_ref[:], preferred_element_type=jnp.float32, precision=lax.Precision.HIGHEST) + b2_ref[:]
    hn = jnp.maximum(hn, 0.0)
    h_out[:] = hn
    bm = jnp.max(hn, axis=0, keepdims=True)
    step = pl.program_id(0)

    @pl.when(step == 0)
    def _():
        c_ref[:] = bm

    @pl.when(step > 0)
    def _():
        c_ref[:] = jnp.maximum(c_ref[:], bm)

    @pl.when(step == pl.num_programs(0) - 1)
    def _():
        c_ref[:] = jnp.maximum(c_ref[:] + maxw_ref[:], 0.0) + EPS


def _layer_mlp(h, dn0, dn1, W1, b1, W2, b2, maxw):
    grid = N // BLK
    w1a, w1b = W1[:64], W1[64:]
    return pl.pallas_call(
        _layer_body,
        grid=(grid,),
        in_specs=[
            pl.BlockSpec((BLK, D), lambda i: (i, 0)),
            pl.BlockSpec((BLK, D), lambda i: (i, 0)),
            pl.BlockSpec((BLK, D), lambda i: (i, 0)),
            pl.BlockSpec((64, 2 * D), lambda i: (0, 0)),
            pl.BlockSpec((64, 2 * D), lambda i: (0, 0)),
            pl.BlockSpec((1, 2 * D), lambda i: (0, 0)),
            pl.BlockSpec((2 * D, D), lambda i: (0, 0)),
            pl.BlockSpec((1, D), lambda i: (0, 0)),
            pl.BlockSpec((1, 1), lambda i: (0, 0)),
        ],
        out_specs=[
            pl.BlockSpec((BLK, D), lambda i: (i, 0)),
            pl.BlockSpec((1, D), lambda i: (0, 0)),
        ],
        out_shape=[
            jax.ShapeDtypeStruct((N, D), jnp.float32),
            jax.ShapeDtypeStruct((1, D), jnp.float32),
        ],
    )(h, dn0, dn1, w1a, w1b, b1.reshape(1, -1), W2,
      b2.reshape(1, -1), maxw)


def _head_body(h_ref, g_ref, b_ref, mw_ref, mb_ref, ow_ref, ob_ref, o_ref):
    hb = h_ref[:]
    mu = jnp.mean(hb, axis=-1, keepdims=True)
    var = jnp.mean((hb - mu) ** 2, axis=-1, keepdims=True)
    hb = (hb - mu) * lax.rsqrt(var + 1e-5) * g_ref[:] + b_ref[:]
    for j in range(3):
        z = jnp.dot(hb, mw_ref[j], preferred_element_type=jnp.float32, precision=lax.Precision.HIGHEST) + mb_ref[j]
        hb = 0.5 * z * (1.0 + lax.erf(z * 0.7071067811865476))
    o = jnp.dot(hb, ow_ref[:], preferred_element_type=jnp.float32, precision=lax.Precision.HIGHEST) + ob_ref[:]
    o_ref[:] = o


def _head(h, ln_g, ln_b, mlp_W, mlp_b, out_W, out_b):
    grid = N // BLK
    return pl.pallas_call(
        _head_body,
        grid=(grid,),
        in_specs=[
            pl.BlockSpec((BLK, D), lambda i: (i, 0)),
            pl.BlockSpec((1, D), lambda i: (0, 0)),
            pl.BlockSpec((1, D), lambda i: (0, 0)),
            pl.BlockSpec((3, D, D), lambda i: (0, 0, 0)),
            pl.BlockSpec((3, 1, D), lambda i: (0, 0, 0)),
            pl.BlockSpec((D, 1), lambda i: (0, 0)),
            pl.BlockSpec((1, 1), lambda i: (0, 0)),
        ],
        out_specs=pl.BlockSpec((BLK, 1), lambda i: (i, 0)),
        out_shape=jax.ShapeDtypeStruct((N, 1), jnp.float32),
    )(h, ln_g.reshape(1, D), ln_b.reshape(1, D), mlp_W,
      mlp_b.reshape(3, 1, D), out_W, out_b.reshape(1, 1))


def _agg_sums_xla(h, src, dst, w, C):
    msg = jnp.maximum(h[src] + w, 0.0) + EPS
    p = jnp.exp(msg - C)
    den = jax.ops.segment_sum(p, dst, num_segments=N)
    num = jax.ops.segment_sum(msg * p, dst, num_segments=N)
    return (den[:, :64], den[:, 64:], num[:, :64], num[:, 64:])


from jax.experimental.pallas import tpu_sc as plsc

NS = 16
CH = 64
NCHUNK = 320
EPT = CH * NCHUNK
E_PAD = NS * EPT
PAD = E_PAD - E
NPADROW = 240
NACC = N + NPADROW
RPT = NACC // NS


def _sc_body(h_hbm, edat, wdat, C2, dn_out,
             rows_v, upd, ebuf, wbuf, cbuf,
             acc, gse0, gse1, pse0, pse1, ise0, ise1, ise2, ise3):
    c = lax.axis_index("c")
    s = lax.axis_index("s")

    pltpu.sync_copy(C2.at[c], cbuf)

    def zbody(r, carry):
        for q in range(8):
            upd[0, r, q * 16:(q + 1) * 16] = jnp.zeros((16,), jnp.float32)
        return carry

    lax.fori_loop(0, CH, zbody, 0)
    for k in range(RPT // CH):
        pltpu.sync_copy(upd.at[0], acc.at[pl.ds(s * RPT + k * CH, CH)])
    plsc.subcore_barrier()

    cvec = tuple(cbuf[q * 16:(q + 1) * 16] for q in range(4))
    gse = (gse0, gse1)
    pse = (pse0, pse1)
    ise = (ise0, ise1, ise2, ise3)

    def fetch_idx(t, j):
        pltpu.async_copy(edat.at[s, t], ebuf.at[j], ise[j])
        pltpu.async_copy(wdat.at[s, t], wbuf.at[j], ise[j])

    def wait_idx(t, j):
        pltpu.make_async_copy(edat.at[s, t], ebuf.at[j], ise[j]).wait()
        pltpu.make_async_copy(wdat.at[s, t], wbuf.at[j], ise[j]).wait()

    def issue_gather(t, j, bb):
        pltpu.async_copy(h_hbm.at[ebuf.at[j, 0]], rows_v.at[bb], gse[bb])

    fetch_idx(0, 0)
    wait_idx(0, 0)
    issue_gather(0, 0, 0)
    fetch_idx(1, 1)

    def gbody(g, cvec):
        for b4 in range(4):
            t = 4 * g + b4
            j = b4
            jn = (b4 + 1) % 4
            j2 = (b4 + 2) % 4
            b = b4 % 2

            pltpu.make_async_copy(h_hbm.at[ebuf.at[j, 0]], rows_v.at[b],
                                  gse[b]).wait()

            @pl.when(t < NCHUNK - 1)
            def _():
                wait_idx(t + 1, jn)
                issue_gather(t + 1, jn, 1 - b)

            @pl.when(t >= 2)
            def _():
                pltpu.make_async_copy(upd.at[b], acc.at[ebuf.at[j2, 1]],
                                      pse[b]).wait()

            @pl.when(t < NCHUNK - 2)
            def _():
                fetch_idx(t + 2, j2)

            c0, c1, c2, c3 = cvec
            half = c * 64

            def ebody(i, carry):
                wv16 = wbuf[j, pl.ds(i * 16, 16)]
                for k in range(16):
                    e = i * 16 + k
                    wv = jnp.full((16,), wv16[k], jnp.float32)
                    for q, cq in enumerate((c0, c1, c2, c3)):
                        hq = rows_v[b, e, pl.ds(half + q * 16, 16)]
                        m = jnp.maximum(hq + wv, 0.0) + EPS
                        p = jnp.exp(m - cq)
                        upd[b, e, q * 16:(q + 1) * 16] = p
                        upd[b, e, 64 + q * 16:64 + (q + 1) * 16] = m * p
                return carry

            lax.fori_loop(0, CH // 16, ebody, 0)

            pltpu.async_copy(upd.at[b], acc.at[ebuf.at[j, 1]], pse[b],
                             add=True)
        return cvec

    lax.fori_loop(0, NCHUNK // 4, gbody, cvec)

    for t in (NCHUNK - 2, NCHUNK - 1):
        pltpu.make_async_copy(upd.at[t % 2], acc.at[ebuf.at[t % 4, 1]],
                              pse[t % 2]).wait()
    plsc.subcore_barrier()

    @pl.when(s < NS - 1)
    def _():
        pltpu.sync_copy(acc.at[pl.ds(s * RPT, RPT)],
                        dn_out.at[c, pl.ds(s * RPT, RPT)])

    @pl.when(s == NS - 1)
    def _():
        last = N - (NS - 1) * RPT
        pltpu.sync_copy(acc.at[pl.ds((NS - 1) * RPT, last)],
                        dn_out.at[c, pl.ds((NS - 1) * RPT, last)])


def _sc_agg(h, edat, wdat, C):
    C2 = C.reshape(2, 64)
    kern = pl.kernel(
        _sc_body,
        out_type=jax.ShapeDtypeStruct((2, N, D), jnp.float32),
        mesh=plsc.VectorSubcoreMesh(core_axis_name="c", subcore_axis_name="s"),
        scratch_types=[
            pltpu.VMEM((2, CH, D), jnp.float32),
            pltpu.VMEM((2, CH, D), jnp.float32),
            pltpu.VMEM((4, 2, CH), jnp.int32),
            pltpu.VMEM((4, CH), jnp.float32),
            pltpu.VMEM((64,), jnp.float32),
            pltpu.VMEM_SHARED((NACC, D), jnp.float32),
            pltpu.SemaphoreType.DMA,
            pltpu.SemaphoreType.DMA,
            pltpu.SemaphoreType.DMA,
            pltpu.SemaphoreType.DMA,
            pltpu.SemaphoreType.DMA,
            pltpu.SemaphoreType.DMA,
            pltpu.SemaphoreType.DMA,
            pltpu.SemaphoreType.DMA,
        ],
    )
    return kern(h, edat, wdat, C2)


def kernel(x, edge_index, edge_weight, emb, W1, b1, W2, b2, ln_g, ln_b,
           mlp_W, mlp_b, out_W, out_b):
    src = edge_index[0].astype(jnp.int32)
    dst = edge_index[1].astype(jnp.int32)
    ar = jnp.arange(PAD, dtype=jnp.int32)
    srcp = jnp.concatenate([src, ar % N]).reshape(NS, NCHUNK, CH)
    dstp = jnp.concatenate([dst, N + (ar % NPADROW)]).reshape(NS, NCHUNK, CH)
    wdat = jnp.concatenate([edge_weight[:, 0],
                            jnp.zeros((PAD,), jnp.float32)]).reshape(
                                NS, NCHUNK, CH)
    edat = jnp.stack([srcp, dstp], axis=2)
    maxw = _maxw(edge_weight)
    h, C = _encode(x, emb, maxw)
    for i in range(L):
        dn = _sc_agg(h, edat, wdat, C)
        h, C = _layer_mlp(h, dn[0], dn[1], W1[i], b1[i], W2[i], b2[i], maxw)
    o = _head(h, ln_g, ln_b, mlp_W, mlp_b, out_W, out_b)
    return o[:, 0]


# speedup vs baseline: 10.2156x; 3.0430x over previous
import functools
import jax
import jax.numpy as jnp
from jax import lax
from jax.experimental import pallas as pl
from jax.experimental.pallas import tpu as pltpu

N = 10000
E = 320000
D = 128
V = 25
L = 6
EPS = 1e-7
BLK = 1000


def _encode_body(x_ref, emb_ref, maxw_ref, h_ref, c_ref):
    xb = x_ref[:]
    a = jnp.zeros((BLK, V), jnp.float32)
    for j in range(3):
        col = xb[:, j:j + 1]
        ids = lax.broadcasted_iota(jnp.int32, (BLK, V), 1)
        a = a + (ids == col).astype(jnp.float32)
    hb = jnp.dot(a, emb_ref[:], preferred_element_type=jnp.float32, precision=lax.Precision.HIGHEST)
    h_ref[:] = hb
    bm = jnp.max(hb, axis=0, keepdims=True)
    step = pl.program_id(0)

    @pl.when(step == 0)
    def _():
        c_ref[:] = bm

    @pl.when(step > 0)
    def _():
        c_ref[:] = jnp.maximum(c_ref[:], bm)

    @pl.when(step == pl.num_programs(0) - 1)
    def _():
        c_ref[:] = jnp.maximum(c_ref[:] + maxw_ref[:], 0.0) + EPS


def _encode(x, emb, maxw):
    grid = N // BLK
    return pl.pallas_call(
        _encode_body,
        grid=(grid,),
        in_specs=[
            pl.BlockSpec((BLK, 3), lambda i: (i, 0)),
            pl.BlockSpec((V, D), lambda i: (0, 0)),
            pl.BlockSpec((1, 1), lambda i: (0, 0)),
        ],
        out_specs=[
            pl.BlockSpec((BLK, D), lambda i: (i, 0)),
            pl.BlockSpec((1, D), lambda i: (0, 0)),
        ],
        out_shape=[
            jax.ShapeDtypeStruct((N, D), jnp.float32),
            jax.ShapeDtypeStruct((1, D), jnp.float32),
        ],
    )(x, emb, maxw)


def _maxw_body(w_ref, o_ref):
    o_ref[:] = jnp.max(w_ref[:]).reshape(1, 1)


def _maxw(ew):
    w2 = ew.reshape(2500, 128)
    return pl.pallas_call(
        _maxw_body,
        out_shape=jax.ShapeDtypeStruct((1, 1), jnp.float32),
    )(w2)


def _layer_body(h_ref, dn0_ref, dn1_ref, w1a_ref, w1b_ref,
                b1_ref, w2_ref, b2_ref, maxw_ref, h_out, c_ref):
    hb = h_ref[:]
    dn0 = dn0_ref[:]
    dn1 = dn1_ref[:]
    agg0 = dn0[:, 64:] / (dn0[:, :64] + 1e-16)
    agg1 = dn1[:, 64:] / (dn1[:, :64] + 1e-16)
    o0 = agg0 + hb[:, :64]
    o1 = agg1 + hb[:, 64:]
    z = (jnp.dot(o0, w1a_ref[:], preferred_element_type=jnp.float32, precision=lax.Precision.HIGHEST)
         + jnp.dot(o1, w1b_ref[:], preferred_element_type=jnp.float32, precision=lax.Precision.HIGHEST)
         + b1_ref[:])
    z = jnp.maximum(z, 0.0)
    hn = jnp.dot(z, w2_ref[:], preferred_element_type=jnp.float32, precision=lax.Precision.HIGHEST) + b2_ref[:]
    hn = jnp.maximum(hn, 0.0)
    h_out[:] = hn
    bm = jnp.max(hn, axis=0, keepdims=True)
    step = pl.program_id(0)

    @pl.when(step == 0)
    def _():
        c_ref[:] = bm

    @pl.when(step > 0)
    def _():
        c_ref[:] = jnp.maximum(c_ref[:], bm)

    @pl.when(step == pl.num_programs(0) - 1)
    def _():
        c_ref[:] = jnp.maximum(c_ref[:] + maxw_ref[:], 0.0) + EPS


def _layer_mlp(h, dn0, dn1, W1, b1, W2, b2, maxw):
    grid = N // BLK
    w1a, w1b = W1[:64], W1[64:]
    return pl.pallas_call(
        _layer_body,
        grid=(grid,),
        in_specs=[
            pl.BlockSpec((BLK, D), lambda i: (i, 0)),
            pl.BlockSpec((BLK, D), lambda i: (i, 0)),
            pl.BlockSpec((BLK, D), lambda i: (i, 0)),
            pl.BlockSpec((64, 2 * D), lambda i: (0, 0)),
            pl.BlockSpec((64, 2 * D), lambda i: (0, 0)),
            pl.BlockSpec((1, 2 * D), lambda i: (0, 0)),
            pl.BlockSpec((2 * D, D), lambda i: (0, 0)),
            pl.BlockSpec((1, D), lambda i: (0, 0)),
            pl.BlockSpec((1, 1), lambda i: (0, 0)),
        ],
        out_specs=[
            pl.BlockSpec((BLK, D), lambda i: (i, 0)),
            pl.BlockSpec((1, D), lambda i: (0, 0)),
        ],
        out_shape=[
            jax.ShapeDtypeStruct((N, D), jnp.float32),
            jax.ShapeDtypeStruct((1, D), jnp.float32),
        ],
    )(h, dn0, dn1, w1a, w1b, b1.reshape(1, -1), W2,
      b2.reshape(1, -1), maxw)


def _head_body(h_ref, g_ref, b_ref, mw_ref, mb_ref, ow_ref, ob_ref, o_ref):
    hb = h_ref[:]
    mu = jnp.mean(hb, axis=-1, keepdims=True)
    var = jnp.mean((hb - mu) ** 2, axis=-1, keepdims=True)
    hb = (hb - mu) * lax.rsqrt(var + 1e-5) * g_ref[:] + b_ref[:]
    for j in range(3):
        z = jnp.dot(hb, mw_ref[j], preferred_element_type=jnp.float32, precision=lax.Precision.HIGHEST) + mb_ref[j]
        hb = 0.5 * z * (1.0 + lax.erf(z * 0.7071067811865476))
    o = jnp.dot(hb, ow_ref[:], preferred_element_type=jnp.float32, precision=lax.Precision.HIGHEST) + ob_ref[:]
    o_ref[:] = o


def _head(h, ln_g, ln_b, mlp_W, mlp_b, out_W, out_b):
    grid = N // BLK
    return pl.pallas_call(
        _head_body,
        grid=(grid,),
        in_specs=[
            pl.BlockSpec((BLK, D), lambda i: (i, 0)),
            pl.BlockSpec((1, D), lambda i: (0, 0)),
            pl.BlockSpec((1, D), lambda i: (0, 0)),
            pl.BlockSpec((3, D, D), lambda i: (0, 0, 0)),
            pl.BlockSpec((3, 1, D), lambda i: (0, 0, 0)),
            pl.BlockSpec((D, 1), lambda i: (0, 0)),
            pl.BlockSpec((1, 1), lambda i: (0, 0)),
        ],
        out_specs=pl.BlockSpec((BLK, 1), lambda i: (i, 0)),
        out_shape=jax.ShapeDtypeStruct((N, 1), jnp.float32),
    )(h, ln_g.reshape(1, D), ln_b.reshape(1, D), mlp_W,
      mlp_b.reshape(3, 1, D), out_W, out_b.reshape(1, 1))


def _agg_sums_xla(h, src, dst, w, C):
    msg = jnp.maximum(h[src] + w, 0.0) + EPS
    p = jnp.exp(msg - C)
    den = jax.ops.segment_sum(p, dst, num_segments=N)
    num = jax.ops.segment_sum(msg * p, dst, num_segments=N)
    return (den[:, :64], den[:, 64:], num[:, :64], num[:, 64:])


from jax.experimental.pallas import tpu_sc as plsc

NS = 16
CH = 64
NCHUNK = 320
EPT = CH * NCHUNK
E_PAD = NS * EPT
PAD = E_PAD - E
NPADROW = 240
NACC = N + NPADROW
RPT = NACC // NS


def _sc_body(h_hbm, edat, wdat, C2, dn_out,
             rows_v, upd, ebuf, wbuf, cbuf,
             acc, gse0, gse1, pse0, pse1, ise0, ise1, ise2, ise3):
    c = lax.axis_index("c")
    s = lax.axis_index("s")

    pltpu.sync_copy(C2.at[c], cbuf)

    def zbody(r, carry):
        for q in range(8):
            upd[0, r, q * 16:(q + 1) * 16] = jnp.zeros((16,), jnp.float32)
        return carry

    lax.fori_loop(0, CH, zbody, 0)
    for k in range(RPT // CH):
        pltpu.sync_copy(upd.at[0], acc.at[pl.ds(s * RPT + k * CH, CH)])
    plsc.subcore_barrier()

    cvec = tuple(cbuf[q * 16:(q + 1) * 16] for q in range(4))
    gse = (gse0, gse1)
    pse = (pse0, pse1)
    ise = (ise0, ise1, ise2, ise3)

    def fetch_idx(t, j):
        pltpu.async_copy(edat.at[s, t], ebuf.at[j], ise[j])
        pltpu.async_copy(wdat.at[s, t], wbuf.at[j], ise[j])

    def wait_idx(t, j):
        pltpu.make_async_copy(edat.at[s, t], ebuf.at[j], ise[j]).wait()
        pltpu.make_async_copy(wdat.at[s, t], wbuf.at[j], ise[j]).wait()

    def issue_gather(t, j, bb):
        pltpu.async_copy(h_hbm.at[ebuf.at[j, 0]], rows_v.at[bb], gse[bb])

    fetch_idx(0, 0)
    wait_idx(0, 0)
    issue_gather(0, 0, 0)
    fetch_idx(1, 1)

    def gbody(g, cvec):
        for b4 in range(4):
            t = 4 * g + b4
            j = b4
            jn = (b4 + 1) % 4
            j2 = (b4 + 2) % 4
            b = b4 % 2

            pltpu.make_async_copy(h_hbm.at[ebuf.at[j, 0]], rows_v.at[b],
                                  gse[b]).wait()

            @pl.when(t < NCHUNK - 1)
            def _():
                wait_idx(t + 1, jn)
                issue_gather(t + 1, jn, 1 - b)

            @pl.when(t >= 2)
            def _():
                pltpu.make_async_copy(upd.at[b], acc.at[ebuf.at[j2, 1]],
                                      pse[b]).wait()

            @pl.when(t < NCHUNK - 2)
            def _():
                fetch_idx(t + 2, j2)

            c0, c1, c2, c3 = cvec

            def make_ebody(half):
                def ebody(i, carry):
                    wv16 = wbuf[j, pl.ds(i * 16, 16)]
                    for k in range(16):
                        e = i * 16 + k
                        wv = jnp.full((16,), wv16[k], jnp.float32)
                        ms = []
                        for q in range(4):
                            hq = rows_v[b, e, half + q * 16:half + (q + 1) * 16]
                            ms.append(jnp.maximum(hq + wv, 0.0) + EPS)
                        ps = [jnp.exp(ms[q] - cq)
                              for q, cq in enumerate((c0, c1, c2, c3))]
                        for q in range(4):
                            upd[b, e, q * 16:(q + 1) * 16] = ps[q]
                            upd[b, e, 64 + q * 16:64 + (q + 1) * 16] = (
                                ms[q] * ps[q])
                    return carry
                return ebody

            @pl.when(c == 0)
            def _():
                lax.fori_loop(0, CH // 16, make_ebody(0), 0)

            @pl.when(c == 1)
            def _():
                lax.fori_loop(0, CH // 16, make_ebody(64), 0)

            pltpu.async_copy(upd.at[b], acc.at[ebuf.at[j, 1]], pse[b],
                             add=True)
        return cvec

    lax.fori_loop(0, NCHUNK // 4, gbody, cvec)

    for t in (NCHUNK - 2, NCHUNK - 1):
        pltpu.make_async_copy(upd.at[t % 2], acc.at[ebuf.at[t % 4, 1]],
                              pse[t % 2]).wait()
    plsc.subcore_barrier()

    @pl.when(s < NS - 1)
    def _():
        pltpu.sync_copy(acc.at[pl.ds(s * RPT, RPT)],
                        dn_out.at[c, pl.ds(s * RPT, RPT)])

    @pl.when(s == NS - 1)
    def _():
        last = N - (NS - 1) * RPT
        pltpu.sync_copy(acc.at[pl.ds((NS - 1) * RPT, last)],
                        dn_out.at[c, pl.ds((NS - 1) * RPT, last)])


def _sc_agg(h, edat, wdat, C):
    C2 = C.reshape(2, 64)
    kern = pl.kernel(
        _sc_body,
        out_type=jax.ShapeDtypeStruct((2, N, D), jnp.float32),
        mesh=plsc.VectorSubcoreMesh(core_axis_name="c", subcore_axis_name="s"),
        scratch_types=[
            pltpu.VMEM((2, CH, D), jnp.float32),
            pltpu.VMEM((2, CH, D), jnp.float32),
            pltpu.VMEM((4, 2, CH), jnp.int32),
            pltpu.VMEM((4, CH), jnp.float32),
            pltpu.VMEM((64,), jnp.float32),
            pltpu.VMEM_SHARED((NACC, D), jnp.float32),
            pltpu.SemaphoreType.DMA,
            pltpu.SemaphoreType.DMA,
            pltpu.SemaphoreType.DMA,
            pltpu.SemaphoreType.DMA,
            pltpu.SemaphoreType.DMA,
            pltpu.SemaphoreType.DMA,
            pltpu.SemaphoreType.DMA,
            pltpu.SemaphoreType.DMA,
        ],
    )
    return kern(h, edat, wdat, C2)


def kernel(x, edge_index, edge_weight, emb, W1, b1, W2, b2, ln_g, ln_b,
           mlp_W, mlp_b, out_W, out_b):
    src = edge_index[0].astype(jnp.int32)
    dst = edge_index[1].astype(jnp.int32)
    ar = jnp.arange(PAD, dtype=jnp.int32)
    srcp = jnp.concatenate([src, ar % N]).reshape(NS, NCHUNK, CH)
    dstp = jnp.concatenate([dst, N + (ar % NPADROW)]).reshape(NS, NCHUNK, CH)
    wdat = jnp.concatenate([edge_weight[:, 0],
                            jnp.zeros((PAD,), jnp.float32)]).reshape(
                                NS, NCHUNK, CH)
    edat = jnp.stack([srcp, dstp], axis=2)
    maxw = _maxw(edge_weight)
    h, C = _encode(x, emb, maxw)
    for i in range(L):
        dn = _sc_agg(h, edat, wdat, C)
        h, C = _layer_mlp(h, dn[0], dn[1], W1[i], b1[i], W2[i], b2[i], maxw)
    o = _head(h, ln_g, ln_b, mlp_W, mlp_b, out_W, out_b)
    return o[:, 0]


# speedup vs baseline: 11.6598x; 1.1414x over previous
import functools
import jax
import jax.numpy as jnp
from jax import lax
from jax.experimental import pallas as pl
from jax.experimental.pallas import tpu as pltpu

N = 10000
E = 320000
D = 128
V = 25
L = 6
EPS = 1e-7
BLK = 1000


def _encode_body(x_ref, emb_ref, maxw_ref, h_ref, c_ref):
    xb = x_ref[:]
    a = jnp.zeros((BLK, V), jnp.float32)
    for j in range(3):
        col = xb[:, j:j + 1]
        ids = lax.broadcasted_iota(jnp.int32, (BLK, V), 1)
        a = a + (ids == col).astype(jnp.float32)
    hb = jnp.dot(a, emb_ref[:], preferred_element_type=jnp.float32, precision=lax.Precision.HIGHEST)
    h_ref[:] = hb
    bm = jnp.max(hb, axis=0, keepdims=True)
    step = pl.program_id(0)

    @pl.when(step == 0)
    def _():
        c_ref[:] = bm

    @pl.when(step > 0)
    def _():
        c_ref[:] = jnp.maximum(c_ref[:], bm)

    @pl.when(step == pl.num_programs(0) - 1)
    def _():
        c_ref[:] = jnp.maximum(c_ref[:] + maxw_ref[:], 0.0)


def _encode(x, emb, maxw):
    grid = N // BLK
    return pl.pallas_call(
        _encode_body,
        grid=(grid,),
        in_specs=[
            pl.BlockSpec((BLK, 3), lambda i: (i, 0)),
            pl.BlockSpec((V, D), lambda i: (0, 0)),
            pl.BlockSpec((1, 1), lambda i: (0, 0)),
        ],
        out_specs=[
            pl.BlockSpec((BLK, D), lambda i: (i, 0)),
            pl.BlockSpec((1, D), lambda i: (0, 0)),
        ],
        out_shape=[
            jax.ShapeDtypeStruct((N, D), jnp.float32),
            jax.ShapeDtypeStruct((1, D), jnp.float32),
        ],
    )(x, emb, maxw)


def _maxw_body(w_ref, o_ref):
    o_ref[:] = jnp.max(w_ref[:]).reshape(1, 1)


def _maxw(ew):
    w2 = ew.reshape(2500, 128)
    return pl.pallas_call(
        _maxw_body,
        out_shape=jax.ShapeDtypeStruct((1, 1), jnp.float32),
    )(w2)


def _layer_body(h_ref, dn0_ref, dn1_ref, w1a_ref, w1b_ref,
                b1_ref, w2_ref, b2_ref, maxw_ref, h_out, c_ref):
    hb = h_ref[:]
    dn0 = dn0_ref[:]
    dn1 = dn1_ref[:]
    agg0 = (dn0[:, 64:] + EPS * dn0[:, :64]) / (dn0[:, :64] + 1e-16)
    agg1 = (dn1[:, 64:] + EPS * dn1[:, :64]) / (dn1[:, :64] + 1e-16)
    o0 = agg0 + hb[:, :64]
    o1 = agg1 + hb[:, 64:]
    z = (jnp.dot(o0, w1a_ref[:], preferred_element_type=jnp.float32, precision=lax.Precision.HIGHEST)
         + jnp.dot(o1, w1b_ref[:], preferred_element_type=jnp.float32, precision=lax.Precision.HIGHEST)
         + b1_ref[:])
    z = jnp.maximum(z, 0.0)
    hn = jnp.dot(z, w2_ref[:], preferred_element_type=jnp.float32, precision=lax.Precision.HIGHEST) + b2_ref[:]
    hn = jnp.maximum(hn, 0.0)
    h_out[:] = hn
    bm = jnp.max(hn, axis=0, keepdims=True)
    step = pl.program_id(0)

    @pl.when(step == 0)
    def _():
        c_ref[:] = bm

    @pl.when(step > 0)
    def _():
        c_ref[:] = jnp.maximum(c_ref[:], bm)

    @pl.when(step == pl.num_programs(0) - 1)
    def _():
        c_ref[:] = jnp.maximum(c_ref[:] + maxw_ref[:], 0.0)


def _layer_mlp(h, dn0, dn1, W1, b1, W2, b2, maxw):
    grid = N // BLK
    w1a, w1b = W1[:64], W1[64:]
    return pl.pallas_call(
        _layer_body,
        grid=(grid,),
        in_specs=[
            pl.BlockSpec((BLK, D), lambda i: (i, 0)),
            pl.BlockSpec((BLK, D), lambda i: (i, 0)),
            pl.BlockSpec((BLK, D), lambda i: (i, 0)),
            pl.BlockSpec((64, 2 * D), lambda i: (0, 0)),
            pl.BlockSpec((64, 2 * D), lambda i: (0, 0)),
            pl.BlockSpec((1, 2 * D), lambda i: (0, 0)),
            pl.BlockSpec((2 * D, D), lambda i: (0, 0)),
            pl.BlockSpec((1, D), lambda i: (0, 0)),
            pl.BlockSpec((1, 1), lambda i: (0, 0)),
        ],
        out_specs=[
            pl.BlockSpec((BLK, D), lambda i: (i, 0)),
            pl.BlockSpec((1, D), lambda i: (0, 0)),
        ],
        out_shape=[
            jax.ShapeDtypeStruct((N, D), jnp.float32),
            jax.ShapeDtypeStruct((1, D), jnp.float32),
        ],
    )(h, dn0, dn1, w1a, w1b, b1.reshape(1, -1), W2,
      b2.reshape(1, -1), maxw)


def _head_body(h_ref, g_ref, b_ref, mw_ref, mb_ref, ow_ref, ob_ref, o_ref):
    hb = h_ref[:]
    mu = jnp.mean(hb, axis=-1, keepdims=True)
    var = jnp.mean((hb - mu) ** 2, axis=-1, keepdims=True)
    hb = (hb - mu) * lax.rsqrt(var + 1e-5) * g_ref[:] + b_ref[:]
    for j in range(3):
        z = jnp.dot(hb, mw_ref[j], preferred_element_type=jnp.float32, precision=lax.Precision.HIGHEST) + mb_ref[j]
        hb = 0.5 * z * (1.0 + lax.erf(z * 0.7071067811865476))
    o = jnp.dot(hb, ow_ref[:], preferred_element_type=jnp.float32, precision=lax.Precision.HIGHEST) + ob_ref[:]
    o_ref[:] = o


def _head(h, ln_g, ln_b, mlp_W, mlp_b, out_W, out_b):
    grid = N // BLK
    return pl.pallas_call(
        _head_body,
        grid=(grid,),
        in_specs=[
            pl.BlockSpec((BLK, D), lambda i: (i, 0)),
            pl.BlockSpec((1, D), lambda i: (0, 0)),
            pl.BlockSpec((1, D), lambda i: (0, 0)),
            pl.BlockSpec((3, D, D), lambda i: (0, 0, 0)),
            pl.BlockSpec((3, 1, D), lambda i: (0, 0, 0)),
            pl.BlockSpec((D, 1), lambda i: (0, 0)),
            pl.BlockSpec((1, 1), lambda i: (0, 0)),
        ],
        out_specs=pl.BlockSpec((BLK, 1), lambda i: (i, 0)),
        out_shape=jax.ShapeDtypeStruct((N, 1), jnp.float32),
    )(h, ln_g.reshape(1, D), ln_b.reshape(1, D), mlp_W,
      mlp_b.reshape(3, 1, D), out_W, out_b.reshape(1, 1))


def _agg_sums_xla(h, src, dst, w, C):
    msg = jnp.maximum(h[src] + w, 0.0) + EPS
    p = jnp.exp(msg - C)
    den = jax.ops.segment_sum(p, dst, num_segments=N)
    num = jax.ops.segment_sum(msg * p, dst, num_segments=N)
    return (den[:, :64], den[:, 64:], num[:, :64], num[:, 64:])


from jax.experimental.pallas import tpu_sc as plsc

NS = 16
CH = 80
NCHUNK = 256
EPT = CH * NCHUNK
E_PAD = NS * EPT
PAD = E_PAD - E
NPADROW = 240
NACC = N + NPADROW
RPT = NACC // NS


def _sc_body(h_hbm, edat, wdat, C2, dn_out,
             rows_v, upd, ebuf, wbuf, cbuf,
             acc, gse0, gse1, pse0, pse1, ise0, ise1, ise2, ise3):
    c = lax.axis_index("c")
    s = lax.axis_index("s")

    pltpu.sync_copy(C2.at[c], cbuf)

    def zbody(r, carry):
        for q in range(8):
            upd[0, r, q * 16:(q + 1) * 16] = jnp.zeros((16,), jnp.float32)
        return carry

    lax.fori_loop(0, CH, zbody, 0)
    for k in range(RPT // CH):
        pltpu.sync_copy(upd.at[0], acc.at[pl.ds(s * RPT + k * CH, CH)])
    plsc.subcore_barrier()

    cvec = tuple(cbuf[q * 16:(q + 1) * 16] for q in range(4))
    gse = (gse0, gse1)
    pse = (pse0, pse1)
    ise = (ise0, ise1, ise2, ise3)

    def fetch_idx(t, j):
        pltpu.async_copy(edat.at[s, t], ebuf.at[j], ise[j])
        pltpu.async_copy(wdat.at[s, t], wbuf.at[j], ise[j])

    def wait_idx(t, j):
        pltpu.make_async_copy(edat.at[s, t], ebuf.at[j], ise[j]).wait()
        pltpu.make_async_copy(wdat.at[s, t], wbuf.at[j], ise[j]).wait()

    def issue_gather(t, j, bb):
        pltpu.async_copy(h_hbm.at[ebuf.at[j, 0]], rows_v.at[bb], gse[bb])

    fetch_idx(0, 0)
    wait_idx(0, 0)
    issue_gather(0, 0, 0)
    fetch_idx(1, 1)

    def gbody(g, cvec):
        for b4 in range(4):
            t = 4 * g + b4
            j = b4
            jn = (b4 + 1) % 4
            j2 = (b4 + 2) % 4
            b = b4 % 2

            pltpu.make_async_copy(h_hbm.at[ebuf.at[j, 0]], rows_v.at[b],
                                  gse[b]).wait()

            @pl.when(t < NCHUNK - 1)
            def _():
                wait_idx(t + 1, jn)
                issue_gather(t + 1, jn, 1 - b)

            @pl.when(t >= 2)
            def _():
                pltpu.make_async_copy(upd.at[b], acc.at[ebuf.at[j2, 1]],
                                      pse[b]).wait()

            @pl.when(t < NCHUNK - 2)
            def _():
                fetch_idx(t + 2, j2)

            c0, c1, c2, c3 = cvec

            def make_ebody(half):
                def ebody(i, carry):
                    wv16 = wbuf[j, pl.ds(i * 16, 16)]
                    for k in range(16):
                        e = i * 16 + k
                        wv = jnp.full((16,), wv16[k], jnp.float32)
                        ms = []
                        for q in range(4):
                            hq = rows_v[b, e, half + q * 16:half + (q + 1) * 16]
                            ms.append(jnp.maximum(hq + wv, 0.0))
                        ps = [jnp.exp(ms[q] - cq)
                              for q, cq in enumerate((c0, c1, c2, c3))]
                        for q in range(4):
                            upd[b, e, q * 16:(q + 1) * 16] = ps[q]
                            upd[b, e, 64 + q * 16:64 + (q + 1) * 16] = (
                                ms[q] * ps[q])
                    return carry
                return ebody

            @pl.when(c == 0)
            def _():
                lax.fori_loop(0, CH // 16, make_ebody(0), 0)

            @pl.when(c == 1)
            def _():
                lax.fori_loop(0, CH // 16, make_ebody(64), 0)

            pltpu.async_copy(upd.at[b], acc.at[ebuf.at[j, 1]], pse[b],
                             add=True)
        return cvec

    lax.fori_loop(0, NCHUNK // 4, gbody, cvec)

    for t in (NCHUNK - 2, NCHUNK - 1):
        pltpu.make_async_copy(upd.at[t % 2], acc.at[ebuf.at[t % 4, 1]],
                              pse[t % 2]).wait()
    plsc.subcore_barrier()

    @pl.when(s < NS - 1)
    def _():
        pltpu.sync_copy(acc.at[pl.ds(s * RPT, RPT)],
                        dn_out.at[c, pl.ds(s * RPT, RPT)])

    @pl.when(s == NS - 1)
    def _():
        last = N - (NS - 1) * RPT
        pltpu.sync_copy(acc.at[pl.ds((NS - 1) * RPT, last)],
                        dn_out.at[c, pl.ds((NS - 1) * RPT, last)])


def _sc_agg(h, edat, wdat, C):
    C2 = C.reshape(2, 64)
    kern = pl.kernel(
        _sc_body,
        out_type=jax.ShapeDtypeStruct((2, N, D), jnp.float32),
        mesh=plsc.VectorSubcoreMesh(core_axis_name="c", subcore_axis_name="s"),
        scratch_types=[
            pltpu.VMEM((2, CH, D), jnp.float32),
            pltpu.VMEM((2, CH, D), jnp.float32),
            pltpu.VMEM((4, 2, CH), jnp.int32),
            pltpu.VMEM((4, CH), jnp.float32),
            pltpu.VMEM((64,), jnp.float32),
            pltpu.VMEM_SHARED((NACC, D), jnp.float32),
            pltpu.SemaphoreType.DMA,
            pltpu.SemaphoreType.DMA,
            pltpu.SemaphoreType.DMA,
            pltpu.SemaphoreType.DMA,
            pltpu.SemaphoreType.DMA,
            pltpu.SemaphoreType.DMA,
            pltpu.SemaphoreType.DMA,
            pltpu.SemaphoreType.DMA,
        ],
    )
    return kern(h, edat, wdat, C2)


def kernel(x, edge_index, edge_weight, emb, W1, b1, W2, b2, ln_g, ln_b,
           mlp_W, mlp_b, out_W, out_b):
    src = edge_index[0].astype(jnp.int32)
    dst = edge_index[1].astype(jnp.int32)
    ar = jnp.arange(PAD, dtype=jnp.int32)
    srcp = jnp.concatenate([src, ar % N]).reshape(NS, NCHUNK, CH)
    dstp = jnp.concatenate([dst, N + (ar % NPADROW)]).reshape(NS, NCHUNK, CH)
    wdat = jnp.concatenate([edge_weight[:, 0],
                            jnp.zeros((PAD,), jnp.float32)]).reshape(
                                NS, NCHUNK, CH)
    edat = jnp.stack([srcp, dstp], axis=2)
    maxw = _maxw(edge_weight)
    h, C = _encode(x, emb, maxw)
    for i in range(L):
        dn = _sc_agg(h, edat, wdat, C)
        h, C = _layer_mlp(h, dn[0], dn[1], W1[i], b1[i], W2[i], b2[i], maxw)
    o = _head(h, ln_g, ln_b, mlp_W, mlp_b, out_W, out_b)
    return o[:, 0]


# speedup vs baseline: 12.2841x; 1.0535x over previous
import functools
import jax
import jax.numpy as jnp
from jax import lax
from jax.experimental import pallas as pl
from jax.experimental.pallas import tpu as pltpu

N = 10000
E = 320000
D = 128
V = 25
L = 6
EPS = 1e-7
BLK = 1000


def _encode_body(x_ref, emb_ref, maxw_ref, h_ref, c_ref):
    xb = x_ref[:]
    a = jnp.zeros((BLK, V), jnp.float32)
    for j in range(3):
        col = xb[:, j:j + 1]
        ids = lax.broadcasted_iota(jnp.int32, (BLK, V), 1)
        a = a + (ids == col).astype(jnp.float32)
    hb = jnp.dot(a, emb_ref[:], preferred_element_type=jnp.float32, precision=lax.Precision.HIGHEST)
    h_ref[:] = hb
    bm = jnp.max(hb, axis=0, keepdims=True)
    step = pl.program_id(0)

    @pl.when(step == 0)
    def _():
        c_ref[:] = bm

    @pl.when(step > 0)
    def _():
        c_ref[:] = jnp.maximum(c_ref[:], bm)

    @pl.when(step == pl.num_programs(0) - 1)
    def _():
        c_ref[:] = jnp.maximum(c_ref[:] + maxw_ref[:], 0.0)


def _encode(x, emb, maxw):
    grid = N // BLK
    return pl.pallas_call(
        _encode_body,
        grid=(grid,),
        in_specs=[
            pl.BlockSpec((BLK, 3), lambda i: (i, 0)),
            pl.BlockSpec((V, D), lambda i: (0, 0)),
            pl.BlockSpec((1, 1), lambda i: (0, 0)),
        ],
        out_specs=[
            pl.BlockSpec((BLK, D), lambda i: (i, 0)),
            pl.BlockSpec((1, D), lambda i: (0, 0)),
        ],
        out_shape=[
            jax.ShapeDtypeStruct((N, D), jnp.float32),
            jax.ShapeDtypeStruct((1, D), jnp.float32),
        ],
    )(x, emb, maxw)


def _maxw_body(w_ref, o_ref):
    o_ref[:] = jnp.max(w_ref[:]).reshape(1, 1)


def _maxw(ew):
    w2 = ew.reshape(2500, 128)
    return pl.pallas_call(
        _maxw_body,
        out_shape=jax.ShapeDtypeStruct((1, 1), jnp.float32),
    )(w2)


def _layer_body(h_ref, dn0_ref, dn1_ref, w1a_ref, w1b_ref,
                b1_ref, w2_ref, b2_ref, maxw_ref, h_out, c_ref):
    hb = h_ref[:]
    dn0 = dn0_ref[:]
    dn1 = dn1_ref[:]
    agg0 = (dn0[:, 64:] + EPS * dn0[:, :64]) / (dn0[:, :64] + 1e-16)
    agg1 = (dn1[:, 64:] + EPS * dn1[:, :64]) / (dn1[:, :64] + 1e-16)
    o0 = agg0 + hb[:, :64]
    o1 = agg1 + hb[:, 64:]
    z = (jnp.dot(o0, w1a_ref[:], preferred_element_type=jnp.float32, precision=lax.Precision.HIGHEST)
         + jnp.dot(o1, w1b_ref[:], preferred_element_type=jnp.float32, precision=lax.Precision.HIGHEST)
         + b1_ref[:])
    z = jnp.maximum(z, 0.0)
    hn = jnp.dot(z, w2_ref[:], preferred_element_type=jnp.float32, precision=lax.Precision.HIGHEST) + b2_ref[:]
    hn = jnp.maximum(hn, 0.0)
    h_out[:] = hn
    bm = jnp.max(hn, axis=0, keepdims=True)
    step = pl.program_id(0)

    @pl.when(step == 0)
    def _():
        c_ref[:] = bm

    @pl.when(step > 0)
    def _():
        c_ref[:] = jnp.maximum(c_ref[:], bm)

    @pl.when(step == pl.num_programs(0) - 1)
    def _():
        c_ref[:] = jnp.maximum(c_ref[:] + maxw_ref[:], 0.0)


def _layer_mlp(h, dn0, dn1, W1, b1, W2, b2, maxw):
    grid = N // BLK
    w1a, w1b = W1[:64], W1[64:]
    return pl.pallas_call(
        _layer_body,
        grid=(grid,),
        in_specs=[
            pl.BlockSpec((BLK, D), lambda i: (i, 0)),
            pl.BlockSpec((BLK, D), lambda i: (i, 0)),
            pl.BlockSpec((BLK, D), lambda i: (i, 0)),
            pl.BlockSpec((64, 2 * D), lambda i: (0, 0)),
            pl.BlockSpec((64, 2 * D), lambda i: (0, 0)),
            pl.BlockSpec((1, 2 * D), lambda i: (0, 0)),
            pl.BlockSpec((2 * D, D), lambda i: (0, 0)),
            pl.BlockSpec((1, D), lambda i: (0, 0)),
            pl.BlockSpec((1, 1), lambda i: (0, 0)),
        ],
        out_specs=[
            pl.BlockSpec((BLK, D), lambda i: (i, 0)),
            pl.BlockSpec((1, D), lambda i: (0, 0)),
        ],
        out_shape=[
            jax.ShapeDtypeStruct((N, D), jnp.float32),
            jax.ShapeDtypeStruct((1, D), jnp.float32),
        ],
    )(h, dn0, dn1, w1a, w1b, b1.reshape(1, -1), W2,
      b2.reshape(1, -1), maxw)


def _head_body(h_ref, g_ref, b_ref, mw_ref, mb_ref, ow_ref, ob_ref, o_ref):
    hb = h_ref[:]
    mu = jnp.mean(hb, axis=-1, keepdims=True)
    var = jnp.mean((hb - mu) ** 2, axis=-1, keepdims=True)
    hb = (hb - mu) * lax.rsqrt(var + 1e-5) * g_ref[:] + b_ref[:]
    for j in range(3):
        z = jnp.dot(hb, mw_ref[j], preferred_element_type=jnp.float32, precision=lax.Precision.HIGHEST) + mb_ref[j]
        hb = 0.5 * z * (1.0 + lax.erf(z * 0.7071067811865476))
    o = jnp.dot(hb, ow_ref[:], preferred_element_type=jnp.float32, precision=lax.Precision.HIGHEST) + ob_ref[:]
    o_ref[:] = o


def _head(h, ln_g, ln_b, mlp_W, mlp_b, out_W, out_b):
    grid = N // BLK
    return pl.pallas_call(
        _head_body,
        grid=(grid,),
        in_specs=[
            pl.BlockSpec((BLK, D), lambda i: (i, 0)),
            pl.BlockSpec((1, D), lambda i: (0, 0)),
            pl.BlockSpec((1, D), lambda i: (0, 0)),
            pl.BlockSpec((3, D, D), lambda i: (0, 0, 0)),
            pl.BlockSpec((3, 1, D), lambda i: (0, 0, 0)),
            pl.BlockSpec((D, 1), lambda i: (0, 0)),
            pl.BlockSpec((1, 1), lambda i: (0, 0)),
        ],
        out_specs=pl.BlockSpec((BLK, 1), lambda i: (i, 0)),
        out_shape=jax.ShapeDtypeStruct((N, 1), jnp.float32),
    )(h, ln_g.reshape(1, D), ln_b.reshape(1, D), mlp_W,
      mlp_b.reshape(3, 1, D), out_W, out_b.reshape(1, 1))


def _agg_sums_xla(h, src, dst, w, C):
    msg = jnp.maximum(h[src] + w, 0.0) + EPS
    p = jnp.exp(msg - C)
    den = jax.ops.segment_sum(p, dst, num_segments=N)
    num = jax.ops.segment_sum(msg * p, dst, num_segments=N)
    return (den[:, :64], den[:, 64:], num[:, :64], num[:, 64:])


from jax.experimental.pallas import tpu_sc as plsc

NS = 16
CH = 80
NCHUNK = 256
EPT = CH * NCHUNK
E_PAD = NS * EPT
PAD = E_PAD - E
NPADROW = 240
NACC = N + NPADROW
RPT = NACC // NS


def _sc_body(h_hbm, edat, wdat, C2, dn_out,
             rows_v, upd, ebuf, wbuf, cbuf,
             acc, gse0, gse1, pse0, pse1, ise0, ise1, ise2, ise3):
    c = lax.axis_index("c")
    s = lax.axis_index("s")

    pltpu.sync_copy(C2.at[c], cbuf)

    def zbody(r, carry):
        for q in range(8):
            upd[0, r, q * 16:(q + 1) * 16] = jnp.zeros((16,), jnp.float32)
        return carry

    lax.fori_loop(0, CH, zbody, 0)
    for k in range(RPT // CH):
        pltpu.sync_copy(upd.at[0], acc.at[pl.ds(s * RPT + k * CH, CH)])
    plsc.subcore_barrier()

    cvec = tuple(cbuf[q * 16:(q + 1) * 16] for q in range(4))
    gse = (gse0, gse1)
    pse = (pse0, pse1)
    ise = (ise0, ise1, ise2, ise3)

    def fetch_idx(t, j):
        pltpu.async_copy(edat.at[s, t], ebuf.at[j], ise[j])
        pltpu.async_copy(wdat.at[s, t], wbuf.at[j], ise[j])

    def wait_idx(t, j):
        pltpu.make_async_copy(edat.at[s, t], ebuf.at[j], ise[j]).wait()
        pltpu.make_async_copy(wdat.at[s, t], wbuf.at[j], ise[j]).wait()

    def issue_gather(t, j, bb):
        pltpu.async_copy(h_hbm.at[ebuf.at[j, 0]], rows_v.at[bb], gse[bb])

    fetch_idx(0, 0)
    wait_idx(0, 0)
    issue_gather(0, 0, 0)
    fetch_idx(1, 1)

    def gbody(g, cvec):
        for b4 in range(4):
            t = 4 * g + b4
            j = b4
            jn = (b4 + 1) % 4
            j2 = (b4 + 2) % 4
            b = b4 % 2

            @pl.when(t < NCHUNK - 1)
            def _():
                wait_idx(t + 1, jn)
                issue_gather(t + 1, jn, 1 - b)

            pltpu.make_async_copy(h_hbm.at[ebuf.at[j, 0]], rows_v.at[b],
                                  gse[b]).wait()

            @pl.when(t >= 2)
            def _():
                pltpu.make_async_copy(upd.at[b], acc.at[ebuf.at[j2, 1]],
                                      pse[b]).wait()

            @pl.when(t < NCHUNK - 2)
            def _():
                fetch_idx(t + 2, j2)

            c0, c1, c2, c3 = cvec

            def make_ebody(half):
                def ebody(i, carry):
                    wv16 = wbuf[j, pl.ds(i * 16, 16)]
                    for k in range(16):
                        e = i * 16 + k
                        wv = jnp.full((16,), wv16[k], jnp.float32)
                        ms = []
                        for q in range(4):
                            hq = rows_v[b, e, half + q * 16:half + (q + 1) * 16]
                            ms.append(jnp.maximum(hq + wv, 0.0))
                        ps = [jnp.exp(ms[q] - cq)
                              for q, cq in enumerate((c0, c1, c2, c3))]
                        for q in range(4):
                            upd[b, e, q * 16:(q + 1) * 16] = ps[q]
                            upd[b, e, 64 + q * 16:64 + (q + 1) * 16] = (
                                ms[q] * ps[q])
                    return carry
                return ebody

            @pl.when(c == 0)
            def _():
                lax.fori_loop(0, CH // 16, make_ebody(0), 0)

            @pl.when(c == 1)
            def _():
                lax.fori_loop(0, CH // 16, make_ebody(64), 0)

            pltpu.async_copy(upd.at[b], acc.at[ebuf.at[j, 1]], pse[b],
                             add=True)
        return cvec

    lax.fori_loop(0, NCHUNK // 4, gbody, cvec)

    for t in (NCHUNK - 2, NCHUNK - 1):
        pltpu.make_async_copy(upd.at[t % 2], acc.at[ebuf.at[t % 4, 1]],
                              pse[t % 2]).wait()
    plsc.subcore_barrier()

    @pl.when(s < NS - 1)
    def _():
        pltpu.sync_copy(acc.at[pl.ds(s * RPT, RPT)],
                        dn_out.at[c, pl.ds(s * RPT, RPT)])

    @pl.when(s == NS - 1)
    def _():
        last = N - (NS - 1) * RPT
        pltpu.sync_copy(acc.at[pl.ds((NS - 1) * RPT, last)],
                        dn_out.at[c, pl.ds((NS - 1) * RPT, last)])


def _sc_agg(h, edat, wdat, C):
    C2 = C.reshape(2, 64)
    kern = pl.kernel(
        _sc_body,
        out_type=jax.ShapeDtypeStruct((2, N, D), jnp.float32),
        mesh=plsc.VectorSubcoreMesh(core_axis_name="c", subcore_axis_name="s"),
        scratch_types=[
            pltpu.VMEM((2, CH, D), jnp.float32),
            pltpu.VMEM((2, CH, D), jnp.float32),
            pltpu.VMEM((4, 2, CH), jnp.int32),
            pltpu.VMEM((4, CH), jnp.float32),
            pltpu.VMEM((64,), jnp.float32),
            pltpu.VMEM_SHARED((NACC, D), jnp.float32),
            pltpu.SemaphoreType.DMA,
            pltpu.SemaphoreType.DMA,
            pltpu.SemaphoreType.DMA,
            pltpu.SemaphoreType.DMA,
            pltpu.SemaphoreType.DMA,
            pltpu.SemaphoreType.DMA,
            pltpu.SemaphoreType.DMA,
            pltpu.SemaphoreType.DMA,
        ],
    )
    return kern(h, edat, wdat, C2)


def kernel(x, edge_index, edge_weight, emb, W1, b1, W2, b2, ln_g, ln_b,
           mlp_W, mlp_b, out_W, out_b):
    src = edge_index[0].astype(jnp.int32)
    dst = edge_index[1].astype(jnp.int32)
    ar = jnp.arange(PAD, dtype=jnp.int32)
    srcp = jnp.concatenate([src, ar % N]).reshape(NS, NCHUNK, CH)
    dstp = jnp.concatenate([dst, N + (ar % NPADROW)]).reshape(NS, NCHUNK, CH)
    wdat = jnp.concatenate([edge_weight[:, 0],
                            jnp.zeros((PAD,), jnp.float32)]).reshape(
                                NS, NCHUNK, CH)
    edat = jnp.stack([srcp, dstp], axis=2)
    maxw = _maxw(edge_weight)
    h, C = _encode(x, emb, maxw)
    for i in range(L):
        dn = _sc_agg(h, edat, wdat, C)
        h, C = _layer_mlp(h, dn[0], dn[1], W1[i], b1[i], W2[i], b2[i], maxw)
    o = _head(h, ln_g, ln_b, mlp_W, mlp_b, out_W, out_b)
    return o[:, 0]


# speedup vs baseline: 12.5056x; 1.0180x over previous
import functools
import jax
import jax.numpy as jnp
from jax import lax
from jax.experimental import pallas as pl
from jax.experimental.pallas import tpu as pltpu

N = 10000
E = 320000
D = 128
V = 25
L = 6
EPS = 1e-7
BLK = 2000


def _encode_body(x_ref, emb_ref, maxw_ref, h_ref, c_ref):
    xb = x_ref[:]
    a = jnp.zeros((BLK, V), jnp.float32)
    for j in range(3):
        col = xb[:, j:j + 1]
        ids = lax.broadcasted_iota(jnp.int32, (BLK, V), 1)
        a = a + (ids == col).astype(jnp.float32)
    hb = jnp.dot(a, emb_ref[:], preferred_element_type=jnp.float32, precision=lax.Precision.HIGHEST)
    h_ref[:] = hb
    bm = jnp.max(hb, axis=0, keepdims=True)
    step = pl.program_id(0)

    @pl.when(step == 0)
    def _():
        c_ref[:] = bm

    @pl.when(step > 0)
    def _():
        c_ref[:] = jnp.maximum(c_ref[:], bm)

    @pl.when(step == pl.num_programs(0) - 1)
    def _():
        c_ref[:] = jnp.maximum(c_ref[:] + maxw_ref[:], 0.0)


def _encode(x, emb, maxw):
    grid = N // BLK
    return pl.pallas_call(
        _encode_body,
        grid=(grid,),
        in_specs=[
            pl.BlockSpec((BLK, 3), lambda i: (i, 0)),
            pl.BlockSpec((V, D), lambda i: (0, 0)),
            pl.BlockSpec((1, 1), lambda i: (0, 0)),
        ],
        out_specs=[
            pl.BlockSpec((BLK, D), lambda i: (i, 0)),
            pl.BlockSpec((1, D), lambda i: (0, 0)),
        ],
        out_shape=[
            jax.ShapeDtypeStruct((N, D), jnp.float32),
            jax.ShapeDtypeStruct((1, D), jnp.float32),
        ],
    )(x, emb, maxw)


def _maxw_body(w_ref, o_ref):
    o_ref[:] = jnp.max(w_ref[:]).reshape(1, 1)


def _maxw(ew):
    w2 = ew.reshape(2500, 128)
    return pl.pallas_call(
        _maxw_body,
        out_shape=jax.ShapeDtypeStruct((1, 1), jnp.float32),
    )(w2)


def _layer_body(h_ref, dn0_ref, dn1_ref, w1a_ref, w1b_ref,
                b1_ref, w2_ref, b2_ref, maxw_ref, h_out, c_ref):
    hb = h_ref[:]
    dn0 = dn0_ref[:]
    dn1 = dn1_ref[:]
    agg0 = (dn0[:, 64:] + EPS * dn0[:, :64]) / (dn0[:, :64] + 1e-16)
    agg1 = (dn1[:, 64:] + EPS * dn1[:, :64]) / (dn1[:, :64] + 1e-16)
    o0 = agg0 + hb[:, :64]
    o1 = agg1 + hb[:, 64:]
    z = (jnp.dot(o0, w1a_ref[:], preferred_element_type=jnp.float32, precision=lax.Precision.HIGHEST)
         + jnp.dot(o1, w1b_ref[:], preferred_element_type=jnp.float32, precision=lax.Precision.HIGHEST)
         + b1_ref[:])
    z = jnp.maximum(z, 0.0)
    hn = jnp.dot(z, w2_ref[:], preferred_element_type=jnp.float32, precision=lax.Precision.HIGHEST) + b2_ref[:]
    hn = jnp.maximum(hn, 0.0)
    h_out[:] = hn
    bm = jnp.max(hn, axis=0, keepdims=True)
    step = pl.program_id(0)

    @pl.when(step == 0)
    def _():
        c_ref[:] = bm

    @pl.when(step > 0)
    def _():
        c_ref[:] = jnp.maximum(c_ref[:], bm)

    @pl.when(step == pl.num_programs(0) - 1)
    def _():
        c_ref[:] = jnp.maximum(c_ref[:] + maxw_ref[:], 0.0)


def _layer_mlp(h, dn0, dn1, W1, b1, W2, b2, maxw):
    grid = N // BLK
    w1a, w1b = W1[:64], W1[64:]
    return pl.pallas_call(
        _layer_body,
        grid=(grid,),
        in_specs=[
            pl.BlockSpec((BLK, D), lambda i: (i, 0)),
            pl.BlockSpec((BLK, D), lambda i: (i, 0)),
            pl.BlockSpec((BLK, D), lambda i: (i, 0)),
            pl.BlockSpec((64, 2 * D), lambda i: (0, 0)),
            pl.BlockSpec((64, 2 * D), lambda i: (0, 0)),
            pl.BlockSpec((1, 2 * D), lambda i: (0, 0)),
            pl.BlockSpec((2 * D, D), lambda i: (0, 0)),
            pl.BlockSpec((1, D), lambda i: (0, 0)),
            pl.BlockSpec((1, 1), lambda i: (0, 0)),
        ],
        out_specs=[
            pl.BlockSpec((BLK, D), lambda i: (i, 0)),
            pl.BlockSpec((1, D), lambda i: (0, 0)),
        ],
        out_shape=[
            jax.ShapeDtypeStruct((N, D), jnp.float32),
            jax.ShapeDtypeStruct((1, D), jnp.float32),
        ],
    )(h, dn0, dn1, w1a, w1b, b1.reshape(1, -1), W2,
      b2.reshape(1, -1), maxw)


def _head_body(h_ref, g_ref, b_ref, mw_ref, mb_ref, ow_ref, ob_ref, o_ref):
    hb = h_ref[:]
    mu = jnp.mean(hb, axis=-1, keepdims=True)
    var = jnp.mean((hb - mu) ** 2, axis=-1, keepdims=True)
    hb = (hb - mu) * lax.rsqrt(var + 1e-5) * g_ref[:] + b_ref[:]
    for j in range(3):
        z = jnp.dot(hb, mw_ref[j], preferred_element_type=jnp.float32, precision=lax.Precision.HIGHEST) + mb_ref[j]
        hb = 0.5 * z * (1.0 + lax.erf(z * 0.7071067811865476))
    o = jnp.dot(hb, ow_ref[:], preferred_element_type=jnp.float32, precision=lax.Precision.HIGHEST) + ob_ref[:]
    o_ref[:] = o


def _head(h, ln_g, ln_b, mlp_W, mlp_b, out_W, out_b):
    grid = N // BLK
    return pl.pallas_call(
        _head_body,
        grid=(grid,),
        in_specs=[
            pl.BlockSpec((BLK, D), lambda i: (i, 0)),
            pl.BlockSpec((1, D), lambda i: (0, 0)),
            pl.BlockSpec((1, D), lambda i: (0, 0)),
            pl.BlockSpec((3, D, D), lambda i: (0, 0, 0)),
            pl.BlockSpec((3, 1, D), lambda i: (0, 0, 0)),
            pl.BlockSpec((D, 1), lambda i: (0, 0)),
            pl.BlockSpec((1, 1), lambda i: (0, 0)),
        ],
        out_specs=pl.BlockSpec((BLK, 1), lambda i: (i, 0)),
        out_shape=jax.ShapeDtypeStruct((N, 1), jnp.float32),
    )(h, ln_g.reshape(1, D), ln_b.reshape(1, D), mlp_W,
      mlp_b.reshape(3, 1, D), out_W, out_b.reshape(1, 1))


def _agg_sums_xla(h, src, dst, w, C):
    msg = jnp.maximum(h[src] + w, 0.0) + EPS
    p = jnp.exp(msg - C)
    den = jax.ops.segment_sum(p, dst, num_segments=N)
    num = jax.ops.segment_sum(msg * p, dst, num_segments=N)
    return (den[:, :64], den[:, 64:], num[:, :64], num[:, 64:])


from jax.experimental.pallas import tpu_sc as plsc

NS = 16
CH = 80
NCHUNK = 256
EPT = CH * NCHUNK
E_PAD = NS * EPT
PAD = E_PAD - E
NPADROW = 240
NACC = N + NPADROW
RPT = NACC // NS


def _sc_body(h_hbm, edat, wdat, C2, dn_out,
             rows_v, upd, ebuf, wbuf, cbuf,
             acc, gse0, gse1, pse0, pse1, ise0, ise1, ise2, ise3):
    c = lax.axis_index("c")
    s = lax.axis_index("s")

    pltpu.sync_copy(C2.at[c], cbuf)

    def zbody(r, carry):
        for q in range(8):
            upd[0, r, q * 16:(q + 1) * 16] = jnp.zeros((16,), jnp.float32)
        return carry

    lax.fori_loop(0, CH, zbody, 0)
    for k in range(RPT // CH):
        pltpu.sync_copy(upd.at[0], acc.at[pl.ds(s * RPT + k * CH, CH)])
    plsc.subcore_barrier()

    cvec = tuple(cbuf[q * 16:(q + 1) * 16] for q in range(4))
    gse = (gse0, gse1)
    pse = (pse0, pse1)
    ise = (ise0, ise1, ise2, ise3)

    def fetch_idx(t, j):
        pltpu.async_copy(edat.at[s, t], ebuf.at[j], ise[j])
        pltpu.async_copy(wdat.at[s, t], wbuf.at[j], ise[j])

    def wait_idx(t, j):
        pltpu.make_async_copy(edat.at[s, t], ebuf.at[j], ise[j]).wait()
        pltpu.make_async_copy(wdat.at[s, t], wbuf.at[j], ise[j]).wait()

    def issue_gather(t, j, bb):
        pltpu.async_copy(h_hbm.at[ebuf.at[j, 0]], rows_v.at[bb], gse[bb])

    fetch_idx(0, 0)
    wait_idx(0, 0)
    issue_gather(0, 0, 0)
    fetch_idx(1, 1)

    def gbody(g, cvec):
        for b4 in range(4):
            t = 4 * g + b4
            j = b4
            jn = (b4 + 1) % 4
            j2 = (b4 + 2) % 4
            b = b4 % 2

            @pl.when(t < NCHUNK - 1)
            def _():
                wait_idx(t + 1, jn)
                issue_gather(t + 1, jn, 1 - b)

            pltpu.make_async_copy(h_hbm.at[ebuf.at[j, 0]], rows_v.at[b],
                                  gse[b]).wait()

            @pl.when(t >= 2)
            def _():
                pltpu.make_async_copy(upd.at[b], acc.at[ebuf.at[j2, 1]],
                                      pse[b]).wait()

            @pl.when(t < NCHUNK - 2)
            def _():
                fetch_idx(t + 2, j2)

            c0, c1, c2, c3 = cvec

            def make_ebody(half):
                def ebody(i, carry):
                    wv16 = wbuf[j, pl.ds(i * 16, 16)]
                    for k in range(16):
                        e = i * 16 + k
                        wv = jnp.full((16,), wv16[k], jnp.float32)
                        ms = []
                        for q in range(4):
                            hq = rows_v[b, e, half + q * 16:half + (q + 1) * 16]
                            ms.append(jnp.maximum(hq + wv, 0.0))
                        ps = [jnp.exp(ms[q] - cq)
                              for q, cq in enumerate((c0, c1, c2, c3))]
                        for q in range(4):
                            upd[b, e, q * 16:(q + 1) * 16] = ps[q]
                            upd[b, e, 64 + q * 16:64 + (q + 1) * 16] = (
                                ms[q] * ps[q])
                    return carry
                return ebody

            @pl.when(c == 0)
            def _():
                lax.fori_loop(0, CH // 16, make_ebody(0), 0)

            @pl.when(c == 1)
            def _():
                lax.fori_loop(0, CH // 16, make_ebody(64), 0)

            pltpu.async_copy(upd.at[b], acc.at[ebuf.at[j, 1]], pse[b],
                             add=True)
        return cvec

    lax.fori_loop(0, NCHUNK // 4, gbody, cvec)

    for t in (NCHUNK - 2, NCHUNK - 1):
        pltpu.make_async_copy(upd.at[t % 2], acc.at[ebuf.at[t % 4, 1]],
                              pse[t % 2]).wait()
    plsc.subcore_barrier()

    @pl.when(s < NS - 1)
    def _():
        pltpu.sync_copy(acc.at[pl.ds(s * RPT, RPT)],
                        dn_out.at[c, pl.ds(s * RPT, RPT)])

    @pl.when(s == NS - 1)
    def _():
        last = N - (NS - 1) * RPT
        pltpu.sync_copy(acc.at[pl.ds((NS - 1) * RPT, last)],
                        dn_out.at[c, pl.ds((NS - 1) * RPT, last)])


def _sc_agg(h, edat, wdat, C):
    C2 = C.reshape(2, 64)
    kern = pl.kernel(
        _sc_body,
        out_type=jax.ShapeDtypeStruct((2, N, D), jnp.float32),
        mesh=plsc.VectorSubcoreMesh(core_axis_name="c", subcore_axis_name="s"),
        scratch_types=[
            pltpu.VMEM((2, CH, D), jnp.float32),
            pltpu.VMEM((2, CH, D), jnp.float32),
            pltpu.VMEM((4, 2, CH), jnp.int32),
            pltpu.VMEM((4, CH), jnp.float32),
            pltpu.VMEM((64,), jnp.float32),
            pltpu.VMEM_SHARED((NACC, D), jnp.float32),
            pltpu.SemaphoreType.DMA,
            pltpu.SemaphoreType.DMA,
            pltpu.SemaphoreType.DMA,
            pltpu.SemaphoreType.DMA,
            pltpu.SemaphoreType.DMA,
            pltpu.SemaphoreType.DMA,
            pltpu.SemaphoreType.DMA,
            pltpu.SemaphoreType.DMA,
        ],
    )
    return kern(h, edat, wdat, C2)


def kernel(x, edge_index, edge_weight, emb, W1, b1, W2, b2, ln_g, ln_b,
           mlp_W, mlp_b, out_W, out_b):
    src = edge_index[0].astype(jnp.int32)
    dst = edge_index[1].astype(jnp.int32)
    ar = jnp.arange(PAD, dtype=jnp.int32)
    srcp = jnp.concatenate([src, ar % N]).reshape(NS, NCHUNK, CH)
    dstp = jnp.concatenate([dst, N + (ar % NPADROW)]).reshape(NS, NCHUNK, CH)
    wdat = jnp.concatenate([edge_weight[:, 0],
                            jnp.zeros((PAD,), jnp.float32)]).reshape(
                                NS, NCHUNK, CH)
    edat = jnp.stack([srcp, dstp], axis=2)
    maxw = _maxw(edge_weight)
    h, C = _encode(x, emb, maxw)
    for i in range(L):
        dn = _sc_agg(h, edat, wdat, C)
        h, C = _layer_mlp(h, dn[0], dn[1], W1[i], b1[i], W2[i], b2[i], maxw)
    o = _head(h, ln_g, ln_b, mlp_W, mlp_b, out_W, out_b)
    return o[:, 0]
